# Initial kernel scaffold; baseline (speedup 1.0000x reference)
import functools

import jax
import jax.numpy as jnp
from jax import lax
from jax.experimental import pallas as pl
from jax.experimental.pallas import tpu as pltpu
from jax.experimental.pallas import tpu_sc as plsc

_N = 10000
_D = 128
_E = 320000
_NP = 10240
_CHUNK = 128
_NC = 2
_NS = 16
_C = _E // _CHUNK
_CPC = _C // _NC
_ROWS_PT = _N // _NS
_RED = _NP // _NS

_f32 = jnp.float32


def _sc_mesh():
  return plsc.VectorSubcoreMesh(core_axis_name="c", subcore_axis_name="s")


def _tile_chunk_count(s):
  q, r = _CPC // _NS, _CPC % _NS
  return jnp.where(s < r, q + 1, q).astype(jnp.int32)


def _deg_body(dst_hbm, out_hbm, deg_v, dst_v, sh, red_v, red_out):
  c = lax.axis_index("c")
  s = lax.axis_index("s")
  zeros16 = jnp.zeros((16,), _f32)
  ones16 = jnp.ones((16,), _f32)

  def zero_body(i, _):
    deg_v[pl.ds(i * 16, 16)] = zeros16
    return 0

  lax.fori_loop(0, _NP // 16, zero_body, 0)

  def edge_body(j, _):
    chunk = c * _CPC + s + j * _NS
    e0 = chunk * _CHUNK
    pltpu.sync_copy(dst_hbm.at[pl.ds(e0, _CHUNK)], dst_v)
    for k in range(_CHUNK // 16):
      idx = dst_v[pl.ds(k * 16, 16)]
      plsc.addupdate_scatter(deg_v, [idx], ones16)
    return 0

  lax.fori_loop(0, _tile_chunk_count(s), edge_body, 0)

  pltpu.sync_copy(deg_v, sh.at[s])
  plsc.subcore_barrier()

  pltpu.sync_copy(sh.at[:, pl.ds(s * _RED, _RED)], red_v)

  def red_body(k, _):
    off = k * 16
    v = red_v[0, pl.ds(off, 16)]
    for r in range(1, _NS):
      v = v + red_v[r, pl.ds(off, 16)]
    red_out[pl.ds(off, 16)] = v
    return 0

  lax.fori_loop(0, _RED // 16, red_body, 0)
  pltpu.sync_copy(red_out, out_hbm.at[c, pl.ds(s * _RED, _RED)])


@jax.jit
def _deg_call(dst):
  return pl.kernel(
      _deg_body,
      out_type=jax.ShapeDtypeStruct((_NC, _NP), _f32),
      mesh=_sc_mesh(),
      scratch_types=[
          pltpu.VMEM((_NP,), _f32),
          pltpu.VMEM((_CHUNK,), jnp.int32),
          pltpu.VMEM_SHARED((_NS, _NP), _f32),
          pltpu.VMEM((_NS, _RED), _f32),
          pltpu.VMEM((_RED,), _f32),
      ],
  )(dst)


def _agg_body(hs_hbm, src_hbm, dst_hbm, out_hbm, acc, src_v, dst_v, rows_v,
              sem):
  c = lax.axis_index("c")
  s = lax.axis_index("s")
  zeros16 = jnp.zeros((16,), _f32)

  def zero_rows(i, _):
    for k in range(_D // 16):
      rows_v[i, pl.ds(k * 16, 16)] = zeros16
    return 0

  lax.fori_loop(0, _CHUNK, zero_rows, 0)
  for t in range(5):
    pltpu.sync_copy(rows_v.at[pl.ds(0, 125)],
                    acc.at[pl.ds(s * _ROWS_PT + t * 125, 125)])
  plsc.subcore_barrier()

  def edge_body(j, _):
    chunk = c * _CPC + s + j * _NS
    e0 = chunk * _CHUNK
    pltpu.sync_copy(src_hbm.at[pl.ds(e0, _CHUNK)], src_v)
    pltpu.sync_copy(dst_hbm.at[pl.ds(e0, _CHUNK)], dst_v)
    pltpu.async_copy(hs_hbm.at[src_v], rows_v, sem).wait()
    pltpu.sync_copy(rows_v, acc.at[dst_v], add=True)
    return 0

  lax.fori_loop(0, _tile_chunk_count(s), edge_body, 0)
  plsc.subcore_barrier()
  pltpu.sync_copy(acc.at[pl.ds(s * _ROWS_PT, _ROWS_PT)],
                  out_hbm.at[c, pl.ds(s * _ROWS_PT, _ROWS_PT)])


@jax.jit
def _agg_call(hs, src, dst):
  return pl.kernel(
      _agg_body,
      out_type=jax.ShapeDtypeStruct((_NC, _N, _D), _f32),
      mesh=_sc_mesh(),
      scratch_types=[
          pltpu.VMEM_SHARED((_N, _D), _f32),
          pltpu.VMEM((_CHUNK,), jnp.int32),
          pltpu.VMEM((_CHUNK,), jnp.int32),
          pltpu.VMEM((_CHUNK, _D), _f32),
          pltpu.SemaphoreType.DMA,
      ],
  )(hs, src, dst)


def _dis_body(degp_ref, dis_ref):
  p = degp_ref[...]
  deg = p[0] + p[1] + 1.0
  dis_ref[...] = lax.rsqrt(deg)


@jax.jit
def _dis_call(degp):
  return pl.pallas_call(
      _dis_body,
      out_shape=jax.ShapeDtypeStruct((_NP,), _f32),
  )(degp)


_R = 1000
_GRID = _N // _R


def _row_spec():
  return pl.BlockSpec((_R, _D), lambda i: (i, 0))


def _full_spec(shape):
  return pl.BlockSpec(shape, lambda i: tuple(0 for _ in shape))


def _k1_body(x_ref, w_ref, dis_ref, hs_ref):
  hs_ref[...] = jnp.dot(x_ref[...], w_ref[...],
                        preferred_element_type=_f32) * dis_ref[...]


@jax.jit
def _k1_call(x, w, dis):
  return pl.pallas_call(
      _k1_body,
      grid=(_GRID,),
      in_specs=[_row_spec(), _full_spec((_D, _D)),
                pl.BlockSpec((_R, 1), lambda i: (i, 0))],
      out_specs=_row_spec(),
      out_shape=jax.ShapeDtypeStruct((_N, _D), _f32),
  )(x, w, dis)


def _k2_body(a0_ref, a1_ref, hs_ref, dis_ref, b_ref, hp_ref, wn_ref,
             hn_ref, hsn_ref):
  s = a0_ref[...] + a1_ref[...] + hs_ref[...]
  pre = s * dis_ref[...] + b_ref[...]
  hn = jnp.maximum(pre, 0.0) + hp_ref[...]
  hn_ref[...] = hn
  hsn_ref[...] = jnp.dot(hn, wn_ref[...],
                         preferred_element_type=_f32) * dis_ref[...]


@jax.jit
def _k2_call(a0, a1, hs, dis, b, hp, wn):
  return pl.pallas_call(
      _k2_body,
      grid=(_GRID,),
      in_specs=[_row_spec(), _row_spec(), _row_spec(),
                pl.BlockSpec((_R, 1), lambda i: (i, 0)),
                _full_spec((1, _D)), _row_spec(), _full_spec((_D, _D))],
      out_specs=[_row_spec(), _row_spec()],
      out_shape=[jax.ShapeDtypeStruct((_N, _D), _f32),
                 jax.ShapeDtypeStruct((_N, _D), _f32)],
  )(a0, a1, hs, dis, b, hp, wn)


def _k3_body(a0_ref, a1_ref, hs_ref, dis_ref, b_ref, hp_ref, hn_ref):
  s = a0_ref[...] + a1_ref[...] + hs_ref[...]
  pre = s * dis_ref[...] + b_ref[...]
  hn_ref[...] = jnp.maximum(pre, 0.0) + hp_ref[...]


@jax.jit
def _k3_call(a0, a1, hs, dis, b, hp):
  return pl.pallas_call(
      _k3_body,
      grid=(_GRID,),
      in_specs=[_row_spec(), _row_spec(), _row_spec(),
                pl.BlockSpec((_R, 1), lambda i: (i, 0)),
                _full_spec((1, _D)), _row_spec()],
      out_specs=_row_spec(),
      out_shape=jax.ShapeDtypeStruct((_N, _D), _f32),
  )(a0, a1, hs, dis, b, hp)


def kernel(x, edge_index, W1, b1, W2, b2, W3, b3):
  src = edge_index[0]
  dst = edge_index[1]

  degp = _deg_call(dst)
  dis1d = _dis_call(degp)
  dis = dis1d[:_N].reshape(_N, 1)

  hs = _k1_call(x, W1, dis)
  h = x
  for (b, wn) in ((b1, W2), (b2, W3)):
    a = _agg_call(hs, src, dst)
    h, hs = _k2_call(a[0], a[1], hs, dis, b.reshape(1, _D), h, wn)
  a = _agg_call(hs, src, dst)
  h = _k3_call(a[0], a[1], hs, dis, b3.reshape(1, _D), h)
  return h


# speedup vs baseline: 12.3013x; 12.3013x over previous
import functools

import jax
import jax.numpy as jnp
from jax import lax
from jax.experimental import pallas as pl
from jax.experimental.pallas import tpu as pltpu
from jax.experimental.pallas import tpu_sc as plsc

_N = 10000
_D = 128
_E = 320000
_NP = 10240
_CHUNK = 128
_NC = 2
_NS = 16
_C = _E // _CHUNK
_CPC = _C // _NC
_ROWS_PT = _N // _NS
_RED = _NP // _NS

_f32 = jnp.float32


def _sc_mesh():
  return plsc.VectorSubcoreMesh(core_axis_name="c", subcore_axis_name="s")


def _tile_chunk_count(s):
  q, r = _CPC // _NS, _CPC % _NS
  return jnp.where(s < r, q + 1, q).astype(jnp.int32)


_DW = 128


def _deg_body(dst_hbm, out_hbm, deg_sh, dst_v, ones_v, zeros_v):
  c = lax.axis_index("c")
  s = lax.axis_index("s")
  zeros16 = jnp.zeros((16,), _f32)
  ones16 = jnp.ones((16,), _f32)

  def fill_body(i, _):
    for k in range(_DW // 16):
      ones_v[i, pl.ds(k * 16, 16)] = ones16
      zeros_v[i, pl.ds(k * 16, 16)] = zeros16
    return 0

  lax.fori_loop(0, _CHUNK, fill_body, 0)
  for t in range(5):
    pltpu.sync_copy(zeros_v,
                    deg_sh.at[pl.ds(s * _RED + t * _CHUNK, _CHUNK)])
  plsc.subcore_barrier()

  def edge_body(j, _):
    chunk = c * _CPC + s + j * _NS
    e0 = chunk * _CHUNK
    pltpu.sync_copy(dst_hbm.at[pl.ds(e0, _CHUNK)], dst_v)
    pltpu.sync_copy(ones_v, deg_sh.at[dst_v], add=True)
    return 0

  lax.fori_loop(0, _tile_chunk_count(s), edge_body, 0)
  plsc.subcore_barrier()
  pltpu.sync_copy(deg_sh.at[pl.ds(s * _RED, _RED)],
                  out_hbm.at[c, pl.ds(s * _RED, _RED)])


@jax.jit
def _deg_call(dst):
  return pl.kernel(
      _deg_body,
      out_type=jax.ShapeDtypeStruct((_NC, _NP, _DW), _f32),
      mesh=_sc_mesh(),
      scratch_types=[
          pltpu.VMEM_SHARED((_NP, _DW), _f32),
          pltpu.VMEM((_CHUNK,), jnp.int32),
          pltpu.VMEM((_CHUNK, _DW), _f32),
          pltpu.VMEM((_CHUNK, _DW), _f32),
      ],
  )(dst)


def _agg_body(hs_hbm, src_hbm, dst_hbm, out_hbm, acc, src_v, dst_v, rows_v,
              sem):
  c = lax.axis_index("c")
  s = lax.axis_index("s")
  zeros16 = jnp.zeros((16,), _f32)

  def zero_rows(i, _):
    for k in range(_D // 16):
      rows_v[i, pl.ds(k * 16, 16)] = zeros16
    return 0

  lax.fori_loop(0, _CHUNK, zero_rows, 0)
  for t in range(5):
    pltpu.sync_copy(rows_v,
                    acc.at[pl.ds(s * _RED + t * _CHUNK, _CHUNK)])
  plsc.subcore_barrier()

  def edge_body(j, _):
    chunk = c * _CPC + s + j * _NS
    e0 = chunk * _CHUNK
    pltpu.sync_copy(src_hbm.at[pl.ds(e0, _CHUNK)], src_v)
    pltpu.sync_copy(dst_hbm.at[pl.ds(e0, _CHUNK)], dst_v)
    pltpu.async_copy(hs_hbm.at[src_v], rows_v, sem).wait()
    pltpu.sync_copy(rows_v, acc.at[dst_v], add=True)
    return 0

  lax.fori_loop(0, _tile_chunk_count(s), edge_body, 0)
  plsc.subcore_barrier()
  pltpu.sync_copy(acc.at[pl.ds(s * _RED, _RED)],
                  out_hbm.at[c, pl.ds(s * _RED, _RED)])


@jax.jit
def _agg_call(hs, src, dst):
  return pl.kernel(
      _agg_body,
      out_type=jax.ShapeDtypeStruct((_NC, _NP, _D), _f32),
      mesh=_sc_mesh(),
      scratch_types=[
          pltpu.VMEM_SHARED((_NP, _D), _f32),
          pltpu.VMEM((_CHUNK,), jnp.int32),
          pltpu.VMEM((_CHUNK,), jnp.int32),
          pltpu.VMEM((_CHUNK, _D), _f32),
          pltpu.SemaphoreType.DMA,
      ],
  )(hs, src, dst)


def _dis_body(degp_ref, dis_ref):
  p = degp_ref[...]
  deg = p[0] + p[1] + 1.0
  dis_ref[...] = lax.rsqrt(deg)


@jax.jit
def _dis_call(degp):
  return pl.pallas_call(
      _dis_body,
      out_shape=jax.ShapeDtypeStruct((_NP,), _f32),
  )(degp)


_R = 1000
_GRID = _N // _R


def _row_spec():
  return pl.BlockSpec((_R, _D), lambda i: (i, 0))


def _full_spec(shape):
  return pl.BlockSpec(shape, lambda i: tuple(0 for _ in shape))


def _k1_body(x_ref, w_ref, dis_ref, hs_ref):
  hs_ref[...] = jnp.dot(x_ref[...], w_ref[...],
                        preferred_element_type=_f32) * dis_ref[...]


@jax.jit
def _k1_call(x, w, dis):
  return pl.pallas_call(
      _k1_body,
      grid=(_GRID,),
      in_specs=[_row_spec(), _full_spec((_D, _D)),
                pl.BlockSpec((_R, 1), lambda i: (i, 0))],
      out_specs=_row_spec(),
      out_shape=jax.ShapeDtypeStruct((_N, _D), _f32),
  )(x, w, dis)


def _agg_spec(core):
  return pl.BlockSpec((1, _R, _D), lambda i, c=core: (c, i, 0))


def _k2_body(a0_ref, a1_ref, hs_ref, dis_ref, b_ref, hp_ref, wn_ref,
             hn_ref, hsn_ref):
  s = a0_ref[0] + a1_ref[0] + hs_ref[...]
  pre = s * dis_ref[...] + b_ref[...]
  hn = jnp.maximum(pre, 0.0) + hp_ref[...]
  hn_ref[...] = hn
  hsn_ref[...] = jnp.dot(hn, wn_ref[...],
                         preferred_element_type=_f32) * dis_ref[...]


@jax.jit
def _k2_call(a, hs, dis, b, hp, wn):
  return pl.pallas_call(
      _k2_body,
      grid=(_GRID,),
      in_specs=[_agg_spec(0), _agg_spec(1), _row_spec(),
                pl.BlockSpec((_R, 1), lambda i: (i, 0)),
                _full_spec((1, _D)), _row_spec(), _full_spec((_D, _D))],
      out_specs=[_row_spec(), _row_spec()],
      out_shape=[jax.ShapeDtypeStruct((_N, _D), _f32),
                 jax.ShapeDtypeStruct((_N, _D), _f32)],
  )(a, a, hs, dis, b, hp, wn)


def _k3_body(a0_ref, a1_ref, hs_ref, dis_ref, b_ref, hp_ref, hn_ref):
  s = a0_ref[0] + a1_ref[0] + hs_ref[...]
  pre = s * dis_ref[...] + b_ref[...]
  hn_ref[...] = jnp.maximum(pre, 0.0) + hp_ref[...]


@jax.jit
def _k3_call(a, hs, dis, b, hp):
  return pl.pallas_call(
      _k3_body,
      grid=(_GRID,),
      in_specs=[_agg_spec(0), _agg_spec(1), _row_spec(),
                pl.BlockSpec((_R, 1), lambda i: (i, 0)),
                _full_spec((1, _D)), _row_spec()],
      out_specs=_row_spec(),
      out_shape=jax.ShapeDtypeStruct((_N, _D), _f32),
  )(a, a, hs, dis, b, hp)


def kernel(x, edge_index, W1, b1, W2, b2, W3, b3):
  src = edge_index[0]
  dst = edge_index[1]

  degp = _deg_call(dst)[:, :, 0]
  dis1d = _dis_call(degp)
  dis = dis1d[:_N].reshape(_N, 1)

  hs = _k1_call(x, W1, dis)
  h = x
  for (b, wn) in ((b1, W2), (b2, W3)):
    a = _agg_call(hs, src, dst)
    h, hs = _k2_call(a, hs, dis, b.reshape(1, _D), h, wn)
  a = _agg_call(hs, src, dst)
  h = _k3_call(a, hs, dis, b3.reshape(1, _D), h)
  return h
